# baseline (speedup 1.0000x reference)
import functools

import jax
import jax.numpy as jnp
from jax import lax
from jax.experimental import pallas as pl
from jax.experimental.pallas import tpu as pltpu
from jax.experimental.pallas import tpu_sc as plsc

SEQ = 2048
EMB = 128
NTYP = 2
LANES = 16

NC, NS = 2, 16
NW = NC * NS
CH = 128


def _comb_body(pos_ref, type_ref, out_ref):
    out_ref[...] = pos_ref[...][None, :, :] + type_ref[...][:, None, :]


def _build_comb(pos_table, type_table):
    comb = pl.pallas_call(
        _comb_body,
        out_shape=jax.ShapeDtypeStruct((NTYP, SEQ, EMB), jnp.float32),
    )(pos_table, type_table)
    return comb.reshape(NTYP * SEQ, EMB)


NBUF = 3


def _make_sc_embed(rows):
    rpw = rows // NW
    nch = rpw // CH
    mesh = plsc.VectorSubcoreMesh(core_axis_name="c", subcore_axis_name="s")

    @functools.partial(
        pl.kernel,
        out_type=jax.ShapeDtypeStruct((rows, EMB), jnp.float32),
        mesh=mesh,
        scratch_types=[
            pltpu.VMEM((nch, CH), jnp.int32),
            pltpu.VMEM((nch, CH), jnp.int32),
            pltpu.VMEM((NBUF, CH, EMB), jnp.float32),
            pltpu.VMEM((NBUF, CH, EMB), jnp.float32),
            pltpu.SemaphoreType.DMA((NBUF,)),
            pltpu.SemaphoreType.DMA((NBUF,)),
            pltpu.SemaphoreType.DMA((NBUF,)),
        ],
    )
    def sc_embed(tok_hbm, typ_hbm, word_hbm, comb_hbm, out_hbm,
                 tok_v, cidx_v, wbuf, cbuf, sem_w, sem_c, sem_s):
        wid = lax.axis_index("s") * NC + lax.axis_index("c")
        base = wid * rpw
        s_base = lax.rem(base, SEQ)

        cbase = wid * nch
        pltpu.sync_copy(tok_hbm.at[pl.ds(cbase, nch)], tok_v)
        pltpu.sync_copy(typ_hbm.at[pl.ds(cbase, nch)], cidx_v)

        for j in range(nch):
            for v in range(CH // LANES):
                sl = pl.ds(v * LANES, LANES)
                s_vec = lax.iota(jnp.int32, LANES) + (
                    s_base + j * CH + v * LANES)
                cidx_v[j, sl] = cidx_v[j, sl] * SEQ + s_vec

        gath = {}

        def start_gather(j):
            slot = j % NBUF
            dw = pltpu.async_copy(
                word_hbm.at[tok_v.at[j]], wbuf.at[slot], sem_w.at[slot])
            dc = pltpu.async_copy(
                comb_hbm.at[cidx_v.at[j]], cbuf.at[slot], sem_c.at[slot])
            gath[j] = (dw, dc)

        for j in range(min(NBUF, nch)):
            start_gather(j)

        stores = {}
        for j in range(nch):
            slot = j % NBUF
            dw, dc = gath.pop(j)
            dw.wait()
            dc.wait()

            def add_row(i, carry):
                for v in range(EMB // LANES):
                    sl = pl.ds(v * LANES, LANES)
                    plsc.addupdate(wbuf.at[slot, i, sl], cbuf[slot, i, sl])
                return carry

            lax.fori_loop(0, CH, add_row, 0)

            stores[j] = pltpu.async_copy(
                wbuf.at[slot], out_hbm.at[pl.ds(base + j * CH, CH)],
                sem_s.at[slot])
            nj = j + NBUF
            if nj < nch:
                stores.pop(j).wait()
                start_gather(nj)

        for j in sorted(stores):
            stores.pop(j).wait()

    return sc_embed


def kernel(input_tokens, input_token_types, word_table, pos_table, type_table):
    batch, seq = input_tokens.shape
    rows = batch * seq
    comb = _build_comb(pos_table, type_table)
    tok2d = input_tokens.reshape(rows // CH, CH).astype(jnp.int32)
    typ2d = input_token_types.reshape(rows // CH, CH).astype(jnp.int32)
    out = _make_sc_embed(rows)(tok2d, typ2d, word_table, comb)
    return out.reshape(batch, seq, EMB)


# speedup vs baseline: 1.0880x; 1.0880x over previous
import functools

import jax
import jax.numpy as jnp
from jax import lax
from jax.experimental import pallas as pl
from jax.experimental.pallas import tpu as pltpu
from jax.experimental.pallas import tpu_sc as plsc

SEQ = 2048
EMB = 128
NTYP = 2
LANES = 16

NC, NS = 2, 16
NW = NC * NS
CH = 128
NBUF = 4


def _comb_body(pos_ref, type_ref, out_ref):
    out_ref[...] = pos_ref[...][None, :, :] + type_ref[...][:, None, :]


def _build_comb(pos_table, type_table):
    comb = pl.pallas_call(
        _comb_body,
        out_shape=jax.ShapeDtypeStruct((NTYP, SEQ, EMB), jnp.float32),
    )(pos_table, type_table)
    return comb.reshape(NTYP * SEQ, EMB)


def _make_sc_embed(batch, seq):
    rows = batch * seq
    rpw = rows // NW
    nch = rpw // CH
    mesh = plsc.VectorSubcoreMesh(core_axis_name="c", subcore_axis_name="s")

    @functools.partial(
        pl.kernel,
        out_type=jax.ShapeDtypeStruct((batch, seq, EMB), jnp.float32),
        mesh=mesh,
        scratch_types=[
            pltpu.VMEM((rpw,), jnp.int32),
            pltpu.VMEM((rpw,), jnp.int32),
            pltpu.VMEM((NBUF, CH, EMB), jnp.float32),
            pltpu.SemaphoreType.DMA((NBUF,)),
            pltpu.SemaphoreType.DMA((NBUF,)),
            pltpu.SemaphoreType.DMA((NBUF,)),
        ],
    )
    def sc_embed(tok_hbm, typ_hbm, word_hbm, comb_hbm, out_hbm,
                 tok_v, cidx_v, buf, sem_c, sem_w, sem_s):
        wid = lax.axis_index("s") * NC + lax.axis_index("c")
        b = wid // (seq // rpw)
        soff = lax.rem(wid, seq // rpw) * rpw

        pltpu.sync_copy(tok_hbm.at[b, pl.ds(soff, rpw)], tok_v)
        pltpu.sync_copy(typ_hbm.at[b, pl.ds(soff, rpw)], cidx_v)

        for v in range(rpw // LANES):
            sl = pl.ds(v * LANES, LANES)
            s_vec = lax.iota(jnp.int32, LANES) + (soff + v * LANES)
            cidx_v[sl] = cidx_v[sl] * SEQ + s_vec

        combs, words, stores = {}, {}, {}

        def start_comb(j):
            slot = j % NBUF
            combs[j] = pltpu.async_copy(
                comb_hbm.at[cidx_v.at[pl.ds(j * CH, CH)]],
                buf.at[slot], sem_c.at[slot])

        for j in range(min(NBUF, nch)):
            start_comb(j)

        for j in range(nch):
            slot = j % NBUF
            combs.pop(j).wait()
            words[j] = pltpu.async_copy(
                word_hbm.at[tok_v.at[pl.ds(j * CH, CH)]],
                buf.at[slot], sem_w.at[slot], add=True)
            if j >= 1:
                ps = (j - 1) % NBUF
                words.pop(j - 1).wait()
                stores[j - 1] = pltpu.async_copy(
                    buf.at[ps], out_hbm.at[b, pl.ds(soff + (j - 1) * CH, CH)],
                    sem_s.at[ps])
            if j >= 2 and j - 2 + NBUF < nch:
                stores.pop(j - 2).wait()
                start_comb(j - 2 + NBUF)

        last = nch - 1
        words.pop(last).wait()
        stores[last] = pltpu.async_copy(
            buf.at[last % NBUF], out_hbm.at[b, pl.ds(soff + last * CH, CH)],
            sem_s.at[last % NBUF])
        for j in sorted(stores):
            stores.pop(j).wait()

    return sc_embed


def kernel(input_tokens, input_token_types, word_table, pos_table, type_table):
    batch, seq = input_tokens.shape
    comb = _build_comb(pos_table, type_table)
    return _make_sc_embed(batch, seq)(
        input_tokens, input_token_types, word_table, comb)


# speedup vs baseline: 1.1164x; 1.0261x over previous
import functools

import jax
import jax.numpy as jnp
from jax import lax
from jax.experimental import pallas as pl
from jax.experimental.pallas import tpu as pltpu
from jax.experimental.pallas import tpu_sc as plsc

SEQ = 2048
EMB = 128
NTYP = 2
LANES = 16

NC, NS = 2, 16
NW = NC * NS
CH = 128
NBUF = 6


def _comb_body(pos_ref, type_ref, out_ref):
    out_ref[...] = pos_ref[...][None, :, :] + type_ref[...][:, None, :]


def _build_comb(pos_table, type_table):
    comb = pl.pallas_call(
        _comb_body,
        out_shape=jax.ShapeDtypeStruct((NTYP, SEQ, EMB), jnp.float32),
    )(pos_table, type_table)
    return comb.reshape(NTYP * SEQ, EMB)


def _make_sc_embed(batch, seq):
    rows = batch * seq
    rpw = rows // NW
    nch = rpw // CH
    mesh = plsc.VectorSubcoreMesh(core_axis_name="c", subcore_axis_name="s")

    @functools.partial(
        pl.kernel,
        out_type=jax.ShapeDtypeStruct((batch, seq, EMB), jnp.float32),
        mesh=mesh,
        scratch_types=[
            pltpu.VMEM((rpw,), jnp.int32),
            pltpu.VMEM((rpw,), jnp.int32),
            pltpu.VMEM((NBUF, CH, EMB), jnp.float32),
            pltpu.SemaphoreType.DMA((NBUF,)),
            pltpu.SemaphoreType.DMA((NBUF,)),
            pltpu.SemaphoreType.DMA((NBUF,)),
        ],
    )
    def sc_embed(tok_hbm, typ_hbm, word_hbm, comb_hbm, out_hbm,
                 tok_v, cidx_v, buf, sem_c, sem_w, sem_s):
        wid = lax.axis_index("s") * NC + lax.axis_index("c")
        b = wid // (seq // rpw)
        soff = lax.rem(wid, seq // rpw) * rpw

        pltpu.sync_copy(tok_hbm.at[b, pl.ds(soff, rpw)], tok_v)
        pltpu.sync_copy(typ_hbm.at[b, pl.ds(soff, rpw)], cidx_v)

        for v in range(rpw // LANES):
            sl = pl.ds(v * LANES, LANES)
            s_vec = lax.iota(jnp.int32, LANES) + (soff + v * LANES)
            cidx_v[sl] = cidx_v[sl] * SEQ + s_vec

        combs, words, stores = {}, {}, {}

        def start_comb(j):
            slot = j % NBUF
            combs[j] = pltpu.async_copy(
                comb_hbm.at[cidx_v.at[pl.ds(j * CH, CH)]],
                buf.at[slot], sem_c.at[slot])

        for j in range(min(NBUF, nch)):
            start_comb(j)

        def start_store(j):
            slot = j % NBUF
            words.pop(j).wait()
            stores[j] = pltpu.async_copy(
                buf.at[slot], out_hbm.at[b, pl.ds(soff + j * CH, CH)],
                sem_s.at[slot])

        for j in range(nch):
            slot = j % NBUF
            combs.pop(j).wait()
            words[j] = pltpu.async_copy(
                word_hbm.at[tok_v.at[pl.ds(j * CH, CH)]],
                buf.at[slot], sem_w.at[slot], add=True)
            if j >= 2:
                start_store(j - 2)
            if j >= 3 and j - 3 + NBUF < nch:
                stores.pop(j - 3).wait()
                start_comb(j - 3 + NBUF)

        for j in sorted(words):
            start_store(j)
        for j in sorted(stores):
            stores.pop(j).wait()

    return sc_embed


def kernel(input_tokens, input_token_types, word_table, pos_table, type_table):
    batch, seq = input_tokens.shape
    comb = _build_comb(pos_table, type_table)
    return _make_sc_embed(batch, seq)(
        input_tokens, input_token_types, word_table, comb)
